# baseline (speedup 1.0000x reference)
import functools

import jax
import jax.numpy as jnp
from jax.experimental import pallas as pl
from jax.experimental.pallas import tpu as pltpu


def _matmul_block(x_ref, m_ref, o_ref):
    o_ref[...] = jax.lax.dot_general(
        x_ref[...],
        m_ref[...],
        dimension_numbers=(((1,), (1,)), ((), ())),
        preferred_element_type=jnp.float32,
    )


@functools.partial(jax.jit, static_argnames=())
def kernel(inputs, targets, mem):
    del targets
    b, f = inputs.shape
    c = mem.shape[0]
    cblk = 2048
    grid = (pl.cdiv(c, cblk),)
    return pl.pallas_call(
        _matmul_block,
        grid=grid,
        in_specs=[
            pl.BlockSpec((b, f), lambda i: (0, 0)),
            pl.BlockSpec((cblk, f), lambda i: (i, 0)),
        ],
        out_specs=pl.BlockSpec((b, cblk), lambda i: (0, i)),
        out_shape=jax.ShapeDtypeStruct((b, c), jnp.float32),
        compiler_params=pltpu.CompilerParams(
            dimension_semantics=("parallel",),
        ),
    )(inputs, mem)
